# baseline (speedup 1.0000x reference)
import functools

import jax
import jax.numpy as jnp
from jax import lax
from jax.experimental import pallas as pl
from jax.experimental.pallas import tpu as pltpu
from jax.experimental.pallas import tpu_sc as plsc

N = 10000
E = 320000
G = 64

_NC = 2
_NS = 16
_K = 128
_CHUNKS = 160
_EDGES_PAD = _NS * _CHUNKS * _K
_ACC_ROWS = 10240
_ROWS_PER_TILE = _ACC_ROWS // _NS


def _make_sc_agg(feat, c0):
    c1 = _CHUNKS - c0
    assert c0 % 2 == 0 and c1 % 2 == 0
    mesh = plsc.VectorSubcoreMesh(core_axis_name="c", subcore_axis_name="s")

    @functools.partial(
        pl.kernel,
        mesh=mesh,
        compiler_params=pltpu.CompilerParams(use_tc_tiling_on_sc=False),
        out_type=jax.ShapeDtypeStruct((_NC, _ACC_ROWS, feat), jnp.bfloat16),
        scratch_types=[
            pltpu.VMEM((_K, feat), jnp.bfloat16),
            pltpu.VMEM((_K, feat), jnp.bfloat16),
            pltpu.VMEM((max(c0, 160 - c0), _K), jnp.int32),
            pltpu.VMEM((max(c0, 160 - c0), _K), jnp.int32),
            pltpu.VMEM_SHARED((_ACC_ROWS, feat), jnp.bfloat16),
            pltpu.SemaphoreType.DMA,
            pltpu.SemaphoreType.DMA,
        ],
    )
    def sc_agg(x_hbm, src_hbm, dst_hbm, zeros_hbm, out_hbm, rows0, rows1,
               sidx_v, didx_v, acc, gs0, gs1):
        c = lax.axis_index("c")
        s = lax.axis_index("s")

        pltpu.sync_copy(zeros_hbm, acc.at[pl.ds(s * _ROWS_PER_TILE, _ROWS_PER_TILE)])
        plsc.subcore_barrier()

        def run(base, nchunks):
            pltpu.sync_copy(src_hbm.at[pl.ds(base, nchunks)],
                            sidx_v.at[pl.ds(0, nchunks)])
            pltpu.sync_copy(dst_hbm.at[pl.ds(base, nchunks)],
                            didx_v.at[pl.ds(0, nchunks)])

            pltpu.async_copy(x_hbm.at[sidx_v.at[0]], rows0, gs0)

            def pair_body(g, carry):
                j0 = 2 * g
                pltpu.async_copy(x_hbm.at[sidx_v.at[j0 + 1]], rows1, gs1)
                pltpu.make_async_copy(x_hbm.at[sidx_v.at[j0]], rows0, gs0).wait()
                pltpu.sync_copy(rows0, acc.at[didx_v.at[j0]], add=True)

                @pl.when(g + 1 < nchunks // 2)
                def _():
                    pltpu.async_copy(x_hbm.at[sidx_v.at[j0 + 2]], rows0, gs0)

                pltpu.make_async_copy(x_hbm.at[sidx_v.at[j0 + 1]], rows1,
                                      gs1).wait()
                pltpu.sync_copy(rows1, acc.at[didx_v.at[j0 + 1]], add=True)
                return carry

            lax.fori_loop(0, nchunks // 2, pair_body, 0)

        @pl.when(c == 0)
        def _():
            run(s * c0, c0)

        @pl.when(c == 1)
        def _():
            run(_NS * c0 + s * c1, c1)

        plsc.subcore_barrier()
        pltpu.sync_copy(
            acc.at[pl.ds(s * _ROWS_PER_TILE, _ROWS_PER_TILE)],
            out_hbm.at[c, pl.ds(s * _ROWS_PER_TILE, _ROWS_PER_TILE)],
        )

    return sc_agg


def _tc_mlp1_body(x_ref, p0_ref, p1_ref, wa_ref, ba_ref, g_ref, be_ref,
                  wb_ref, bb_ref, scale_ref, out_ref, outb_ref):
    agg = (p0_ref[0] + p1_ref[0]).astype(jnp.float32)
    z = scale_ref[0, 0] * x_ref[...] + agg
    h = jnp.dot(z, wa_ref[...], preferred_element_type=jnp.float32) + ba_ref[...]
    mu = jnp.mean(h, axis=0, keepdims=True)
    d = h - mu
    var = jnp.mean(d * d, axis=0, keepdims=True)
    hn = d * lax.rsqrt(var + 1e-5) * g_ref[...] + be_ref[...]
    hr = jnp.maximum(hn, 0.0)
    o = jnp.dot(hr, wb_ref[...], preferred_element_type=jnp.float32) + bb_ref[...]
    out_ref[...] = o
    outb_ref[...] = o.astype(jnp.bfloat16)


def _tc_mlp1(fin, fout, x, parts, wa, ba, gamma, beta, wb, bb, scale):
    def part_spec(i):
        return pl.BlockSpec((1, N, fin), lambda g, i=i: (i, 0, 0))
    return pl.pallas_call(
        _tc_mlp1_body,
        grid=(1,),
        out_shape=[jax.ShapeDtypeStruct((N, fout), jnp.float32),
                   jax.ShapeDtypeStruct((N, fout), jnp.bfloat16)],
        in_specs=[
            pl.BlockSpec((N, fin), lambda g: (0, 0)),
            part_spec(0),
            part_spec(1),
            pl.BlockSpec(wa.shape, lambda g: (0, 0)),
            pl.BlockSpec(ba.shape, lambda g: (0, 0)),
            pl.BlockSpec(gamma.shape, lambda g: (0, 0)),
            pl.BlockSpec(beta.shape, lambda g: (0, 0)),
            pl.BlockSpec(wb.shape, lambda g: (0, 0)),
            pl.BlockSpec(bb.shape, lambda g: (0, 0)),
            pl.BlockSpec(memory_space=pltpu.SMEM),
        ],
        out_specs=[pl.BlockSpec((N, fout), lambda g: (0, 0)),
                   pl.BlockSpec((N, fout), lambda g: (0, 0))],
    )(x, parts, parts, wa, ba, gamma, beta, wb, bb, scale)


def _tc_mlp2_pool_body(x_ref, p0_ref, p1_ref, wa_ref, ba_ref, g_ref, be_ref,
                       wb_ref, bb_ref, batch_ref, wlin_ref, blin_ref,
                       scale_ref, out_ref):
    agg = (p0_ref[0] + p1_ref[0]).astype(jnp.float32)
    z = scale_ref[0, 0] * x_ref[...] + agg
    h = jnp.dot(z, wa_ref[...], preferred_element_type=jnp.float32) + ba_ref[...]
    mu = jnp.mean(h, axis=0, keepdims=True)
    d = h - mu
    var = jnp.mean(d * d, axis=0, keepdims=True)
    hn = d * lax.rsqrt(var + 1e-5) * g_ref[...] + be_ref[...]
    hr = jnp.maximum(hn, 0.0)
    h2 = jnp.dot(hr, wb_ref[...], preferred_element_type=jnp.float32) + bb_ref[...]
    gid = lax.broadcasted_iota(jnp.int32, (G, N), 0)
    oh = (gid == batch_ref[...]).astype(jnp.float32)
    pooled = jnp.dot(oh, h2, preferred_element_type=jnp.float32)
    counts = jnp.sum(oh, axis=1, keepdims=True)
    pm = pooled / jnp.maximum(counts, 1.0)
    logits = jnp.dot(pm, wlin_ref[...], preferred_element_type=jnp.float32)
    out_ref[...] = jax.nn.sigmoid(logits + blin_ref[...])


def _tc_mlp2_pool(fin, x, parts, wa, ba, gamma, beta, wb, bb, batch2d,
                  wlin, blin, scale):
    def part_spec(i):
        return pl.BlockSpec((1, N, fin), lambda g, i=i: (i, 0, 0))
    return pl.pallas_call(
        _tc_mlp2_pool_body,
        grid=(1,),
        out_shape=jax.ShapeDtypeStruct((G, 1), jnp.float32),
        in_specs=[
            pl.BlockSpec((N, fin), lambda g: (0, 0)),
            part_spec(0),
            part_spec(1),
            pl.BlockSpec(wa.shape, lambda g: (0, 0)),
            pl.BlockSpec(ba.shape, lambda g: (0, 0)),
            pl.BlockSpec(gamma.shape, lambda g: (0, 0)),
            pl.BlockSpec(beta.shape, lambda g: (0, 0)),
            pl.BlockSpec(wb.shape, lambda g: (0, 0)),
            pl.BlockSpec(bb.shape, lambda g: (0, 0)),
            pl.BlockSpec((1, N), lambda g: (0, 0)),
            pl.BlockSpec(wlin.shape, lambda g: (0, 0)),
            pl.BlockSpec(blin.shape, lambda g: (0, 0)),
            pl.BlockSpec(memory_space=pltpu.SMEM),
        ],
        out_specs=pl.BlockSpec((G, 1), lambda g: (0, 0)),
    )(x, parts, parts, wa, ba, gamma, beta, wb, bb, batch2d, wlin, blin, scale)


_sc_agg_128 = _make_sc_agg(128, 80)
_sc_agg_32 = _make_sc_agg(32, 80)


def kernel(x, edge_index, batch, W1a, b1a, gamma1, beta1, W1b, b1b, eps1,
           W2a, b2a, gamma2, beta2, W2b, b2b, eps2, Wlin, blin):
    src = edge_index[0]
    dst = edge_index[1]
    pad = _EDGES_PAD - E
    srcp = jnp.concatenate([src, jnp.zeros((pad,), jnp.int32)]).reshape(
        _NS * _CHUNKS, _K)
    scrap = N + (jnp.arange(pad, dtype=jnp.int32) % (_ACC_ROWS - N))
    dstp = jnp.concatenate([dst, scrap]).reshape(_NS * _CHUNKS, _K)

    batch2d = batch.reshape(1, N)
    se1 = (1.0 + eps1).reshape(1, 1).astype(jnp.float32)
    se2 = (1.0 + eps2).reshape(1, 1).astype(jnp.float32)

    xb = x.astype(jnp.bfloat16)
    z128 = jnp.zeros((_ROWS_PER_TILE, 128), jnp.bfloat16)
    z32 = jnp.zeros((_ROWS_PER_TILE, 32), jnp.bfloat16)
    parts1 = _sc_agg_128(xb, srcp, dstp, z128)
    h1, h1b = _tc_mlp1(128, 32, x, parts1,
                       W1a, b1a.reshape(1, -1), gamma1.reshape(1, -1),
                       beta1.reshape(1, -1), W1b, b1b.reshape(1, -1), se1)
    parts2 = _sc_agg_32(h1b, srcp, dstp, z32)
    return _tc_mlp2_pool(32, h1, parts2,
                         W2a, b2a.reshape(1, -1), gamma2.reshape(1, -1),
                         beta2.reshape(1, -1), W2b, b2b.reshape(1, -1),
                         batch2d, Wlin, blin.reshape(1, 1), se2)


# speedup vs baseline: 1.4531x; 1.4531x over previous
import functools

import jax
import jax.numpy as jnp
from jax import lax
from jax.experimental import pallas as pl
from jax.experimental.pallas import tpu as pltpu
from jax.experimental.pallas import tpu_sc as plsc

N = 10000
E = 320000
G = 64

_NC = 2
_NS = 16
_K = 80
_CHUNKS = 250
_ACC_ROWS = 10240
_ROWS_PER_TILE = _ACC_ROWS // _NS


def _make_sc_agg(feat, c0):
    c1 = _CHUNKS - c0
    assert c0 % 2 == 0 and c1 % 2 == 0
    mesh = plsc.VectorSubcoreMesh(core_axis_name="c", subcore_axis_name="s")

    @functools.partial(
        pl.kernel,
        mesh=mesh,
        compiler_params=pltpu.CompilerParams(use_tc_tiling_on_sc=False),
        out_type=jax.ShapeDtypeStruct((_NC, _ACC_ROWS, feat), jnp.bfloat16),
        scratch_types=[
            pltpu.VMEM((_K, feat), jnp.bfloat16),
            pltpu.VMEM((_K, feat), jnp.bfloat16),
            pltpu.VMEM((max(c0, _CHUNKS - c0), _K), jnp.int32),
            pltpu.VMEM((max(c0, _CHUNKS - c0), _K), jnp.int32),
            pltpu.VMEM_SHARED((_ACC_ROWS, feat), jnp.bfloat16),
            pltpu.SemaphoreType.DMA,
            pltpu.SemaphoreType.DMA,
        ],
    )
    def sc_agg(x_hbm, src_hbm, dst_hbm, zeros_hbm, out_hbm, rows0, rows1,
               sidx_v, didx_v, acc, gs0, gs1):
        c = lax.axis_index("c")
        s = lax.axis_index("s")

        pltpu.sync_copy(zeros_hbm, acc.at[pl.ds(s * _ROWS_PER_TILE, _ROWS_PER_TILE)])
        plsc.subcore_barrier()

        def run(base, nchunks):
            pltpu.sync_copy(src_hbm.at[pl.ds(base, nchunks)],
                            sidx_v.at[pl.ds(0, nchunks)])
            pltpu.sync_copy(dst_hbm.at[pl.ds(base, nchunks)],
                            didx_v.at[pl.ds(0, nchunks)])

            pltpu.async_copy(x_hbm.at[sidx_v.at[0]], rows0, gs0)

            def pair_body(g, carry):
                j0 = 2 * g
                pltpu.async_copy(x_hbm.at[sidx_v.at[j0 + 1]], rows1, gs1)
                pltpu.make_async_copy(x_hbm.at[sidx_v.at[j0]], rows0, gs0).wait()
                pltpu.sync_copy(rows0, acc.at[didx_v.at[j0]], add=True)

                @pl.when(g + 1 < nchunks // 2)
                def _():
                    pltpu.async_copy(x_hbm.at[sidx_v.at[j0 + 2]], rows0, gs0)

                pltpu.make_async_copy(x_hbm.at[sidx_v.at[j0 + 1]], rows1,
                                      gs1).wait()
                pltpu.sync_copy(rows1, acc.at[didx_v.at[j0 + 1]], add=True)
                return carry

            lax.fori_loop(0, nchunks // 2, pair_body, 0)

        @pl.when(c == 0)
        def _():
            run(s * c0, c0)

        @pl.when(c == 1)
        def _():
            run(_NS * c0 + s * c1, c1)

        plsc.subcore_barrier()
        pltpu.sync_copy(
            acc.at[pl.ds(s * _ROWS_PER_TILE, _ROWS_PER_TILE)],
            out_hbm.at[c, pl.ds(s * _ROWS_PER_TILE, _ROWS_PER_TILE)],
        )

    return sc_agg


def _tc_mlp1_body(x_ref, p0_ref, p1_ref, wa_ref, ba_ref, g_ref, be_ref,
                  wb_ref, bb_ref, scale_ref, out_ref, outb_ref):
    agg = (p0_ref[0] + p1_ref[0]).astype(jnp.float32)
    z = scale_ref[0, 0] * x_ref[...] + agg
    h = jnp.dot(z, wa_ref[...], preferred_element_type=jnp.float32) + ba_ref[...]
    mu = jnp.mean(h, axis=0, keepdims=True)
    d = h - mu
    var = jnp.mean(d * d, axis=0, keepdims=True)
    hn = d * lax.rsqrt(var + 1e-5) * g_ref[...] + be_ref[...]
    hr = jnp.maximum(hn, 0.0)
    o = jnp.dot(hr, wb_ref[...], preferred_element_type=jnp.float32) + bb_ref[...]
    out_ref[...] = o
    outb_ref[...] = o.astype(jnp.bfloat16)


def _tc_mlp1(fin, fout, x, parts, wa, ba, gamma, beta, wb, bb, scale):
    def part_spec(i):
        return pl.BlockSpec((1, N, fin), lambda g, i=i: (i, 0, 0))
    return pl.pallas_call(
        _tc_mlp1_body,
        grid=(1,),
        out_shape=[jax.ShapeDtypeStruct((N, fout), jnp.float32),
                   jax.ShapeDtypeStruct((N, fout), jnp.bfloat16)],
        in_specs=[
            pl.BlockSpec((N, fin), lambda g: (0, 0)),
            part_spec(0),
            part_spec(1),
            pl.BlockSpec(wa.shape, lambda g: (0, 0)),
            pl.BlockSpec(ba.shape, lambda g: (0, 0)),
            pl.BlockSpec(gamma.shape, lambda g: (0, 0)),
            pl.BlockSpec(beta.shape, lambda g: (0, 0)),
            pl.BlockSpec(wb.shape, lambda g: (0, 0)),
            pl.BlockSpec(bb.shape, lambda g: (0, 0)),
            pl.BlockSpec(memory_space=pltpu.SMEM),
        ],
        out_specs=[pl.BlockSpec((N, fout), lambda g: (0, 0)),
                   pl.BlockSpec((N, fout), lambda g: (0, 0))],
    )(x, parts, parts, wa, ba, gamma, beta, wb, bb, scale)


def _tc_mlp2_pool_body(x_ref, p0_ref, p1_ref, wa_ref, ba_ref, g_ref, be_ref,
                       wb_ref, bb_ref, batch_ref, wlin_ref, blin_ref,
                       scale_ref, out_ref):
    agg = (p0_ref[0] + p1_ref[0]).astype(jnp.float32)
    z = scale_ref[0, 0] * x_ref[...] + agg
    h = jnp.dot(z, wa_ref[...], preferred_element_type=jnp.float32) + ba_ref[...]
    mu = jnp.mean(h, axis=0, keepdims=True)
    d = h - mu
    var = jnp.mean(d * d, axis=0, keepdims=True)
    hn = d * lax.rsqrt(var + 1e-5) * g_ref[...] + be_ref[...]
    hr = jnp.maximum(hn, 0.0)
    h2 = jnp.dot(hr, wb_ref[...], preferred_element_type=jnp.float32) + bb_ref[...]
    gid = lax.broadcasted_iota(jnp.int32, (G, N), 0)
    oh = (gid == batch_ref[...]).astype(jnp.float32)
    pooled = jnp.dot(oh, h2, preferred_element_type=jnp.float32)
    counts = jnp.sum(oh, axis=1, keepdims=True)
    pm = pooled / jnp.maximum(counts, 1.0)
    logits = jnp.dot(pm, wlin_ref[...], preferred_element_type=jnp.float32)
    out_ref[...] = jax.nn.sigmoid(logits + blin_ref[...])


def _tc_mlp2_pool(fin, x, parts, wa, ba, gamma, beta, wb, bb, batch2d,
                  wlin, blin, scale):
    def part_spec(i):
        return pl.BlockSpec((1, N, fin), lambda g, i=i: (i, 0, 0))
    return pl.pallas_call(
        _tc_mlp2_pool_body,
        grid=(1,),
        out_shape=jax.ShapeDtypeStruct((G, 1), jnp.float32),
        in_specs=[
            pl.BlockSpec((N, fin), lambda g: (0, 0)),
            part_spec(0),
            part_spec(1),
            pl.BlockSpec(wa.shape, lambda g: (0, 0)),
            pl.BlockSpec(ba.shape, lambda g: (0, 0)),
            pl.BlockSpec(gamma.shape, lambda g: (0, 0)),
            pl.BlockSpec(beta.shape, lambda g: (0, 0)),
            pl.BlockSpec(wb.shape, lambda g: (0, 0)),
            pl.BlockSpec(bb.shape, lambda g: (0, 0)),
            pl.BlockSpec((1, N), lambda g: (0, 0)),
            pl.BlockSpec(wlin.shape, lambda g: (0, 0)),
            pl.BlockSpec(blin.shape, lambda g: (0, 0)),
            pl.BlockSpec(memory_space=pltpu.SMEM),
        ],
        out_specs=pl.BlockSpec((G, 1), lambda g: (0, 0)),
    )(x, parts, parts, wa, ba, gamma, beta, wb, bb, batch2d, wlin, blin, scale)


_sc_agg_128 = _make_sc_agg(128, 180)
_sc_agg_32 = _make_sc_agg(32, 148)


def kernel(x, edge_index, batch, W1a, b1a, gamma1, beta1, W1b, b1b, eps1,
           W2a, b2a, gamma2, beta2, W2b, b2b, eps2, Wlin, blin):
    srcp = edge_index[0].reshape(_NS * _CHUNKS, _K)
    dstp = edge_index[1].reshape(_NS * _CHUNKS, _K)

    batch2d = batch.reshape(1, N)
    se1 = (1.0 + eps1).reshape(1, 1).astype(jnp.float32)
    se2 = (1.0 + eps2).reshape(1, 1).astype(jnp.float32)

    xb = x.astype(jnp.bfloat16)
    z128 = jnp.zeros((_ROWS_PER_TILE, 128), jnp.bfloat16)
    z32 = jnp.zeros((_ROWS_PER_TILE, 32), jnp.bfloat16)
    parts1 = _sc_agg_128(xb, srcp, dstp, z128)
    h1, h1b = _tc_mlp1(128, 32, x, parts1,
                       W1a, b1a.reshape(1, -1), gamma1.reshape(1, -1),
                       beta1.reshape(1, -1), W1b, b1b.reshape(1, -1), se1)
    parts2 = _sc_agg_32(h1b, srcp, dstp, z32)
    return _tc_mlp2_pool(32, h1, parts2,
                         W2a, b2a.reshape(1, -1), gamma2.reshape(1, -1),
                         beta2.reshape(1, -1), W2b, b2b.reshape(1, -1),
                         batch2d, Wlin, blin.reshape(1, 1), se2)


# speedup vs baseline: 1.6476x; 1.1339x over previous
import functools

import jax
import jax.numpy as jnp
from jax import lax
from jax.experimental import pallas as pl
from jax.experimental.pallas import tpu as pltpu
from jax.experimental.pallas import tpu_sc as plsc

N = 10000
E = 320000
G = 64

_NC = 2
_NS = 16
_K = 80
_CHUNKS = 250
_ACC_ROWS = 10240
_ROWS_PER_TILE = _ACC_ROWS // _NS


def _make_sc_agg(feat, c0):
    c1 = _CHUNKS - c0
    assert c0 % 2 == 0 and c1 % 2 == 0
    mesh = plsc.VectorSubcoreMesh(core_axis_name="c", subcore_axis_name="s")

    @functools.partial(
        pl.kernel,
        mesh=mesh,
        compiler_params=pltpu.CompilerParams(use_tc_tiling_on_sc=False),
        out_type=jax.ShapeDtypeStruct((_NC, _ACC_ROWS, feat), jnp.bfloat16),
        scratch_types=[
            pltpu.VMEM((_K, feat), jnp.bfloat16),
            pltpu.VMEM((_K, feat), jnp.bfloat16),
            pltpu.VMEM((max(c0, _CHUNKS - c0), _K), jnp.int32),
            pltpu.VMEM((max(c0, _CHUNKS - c0), _K), jnp.int32),
            pltpu.VMEM_SHARED((_ACC_ROWS, feat), jnp.bfloat16),
            pltpu.SemaphoreType.DMA,
            pltpu.SemaphoreType.DMA,
        ],
    )
    def sc_agg(x_hbm, src_hbm, dst_hbm, zeros_hbm, out_hbm, rows0, rows1,
               sidx_v, didx_v, acc, gs0, gs1):
        c = lax.axis_index("c")
        s = lax.axis_index("s")

        pltpu.sync_copy(zeros_hbm, acc.at[pl.ds(s * _ROWS_PER_TILE, _ROWS_PER_TILE)])
        plsc.subcore_barrier()

        def run(base, nchunks):
            pltpu.sync_copy(src_hbm.at[pl.ds(base, nchunks)],
                            sidx_v.at[pl.ds(0, nchunks)])
            pltpu.sync_copy(dst_hbm.at[pl.ds(base, nchunks)],
                            didx_v.at[pl.ds(0, nchunks)])

            pltpu.async_copy(x_hbm.at[sidx_v.at[0]], rows0, gs0)

            def pair_body(g, carry):
                j0 = 2 * g
                pltpu.async_copy(x_hbm.at[sidx_v.at[j0 + 1]], rows1, gs1)
                pltpu.make_async_copy(x_hbm.at[sidx_v.at[j0]], rows0, gs0).wait()
                pltpu.sync_copy(rows0, acc.at[didx_v.at[j0]], add=True)

                @pl.when(g + 1 < nchunks // 2)
                def _():
                    pltpu.async_copy(x_hbm.at[sidx_v.at[j0 + 2]], rows0, gs0)

                pltpu.make_async_copy(x_hbm.at[sidx_v.at[j0 + 1]], rows1,
                                      gs1).wait()
                pltpu.sync_copy(rows1, acc.at[didx_v.at[j0 + 1]], add=True)
                return carry

            lax.fori_loop(0, nchunks // 2, pair_body, 0)

        @pl.when(c == 0)
        def _():
            run(s * c0, c0)

        @pl.when(c == 1)
        def _():
            run(_NS * c0 + s * c1, c1)

        plsc.subcore_barrier()
        pltpu.sync_copy(
            acc.at[pl.ds(s * _ROWS_PER_TILE, _ROWS_PER_TILE)],
            out_hbm.at[c, pl.ds(s * _ROWS_PER_TILE, _ROWS_PER_TILE)],
        )

    return sc_agg


def _tc_mlp1_body(x_ref, p0_ref, p1_ref, wa_ref, ba_ref, g_ref, be_ref,
                  wb_ref, bb_ref, scale_ref, out_ref, outb_ref):
    agg = (p0_ref[0] + p1_ref[0]).astype(jnp.float32)
    z = scale_ref[0, 0] * x_ref[...] + agg
    h = jnp.dot(z, wa_ref[...], preferred_element_type=jnp.float32) + ba_ref[...]
    mu = jnp.mean(h, axis=0, keepdims=True)
    d = h - mu
    var = jnp.mean(d * d, axis=0, keepdims=True)
    hn = d * lax.rsqrt(var + 1e-5) * g_ref[...] + be_ref[...]
    hr = jnp.maximum(hn, 0.0)
    o = jnp.dot(hr, wb_ref[...], preferred_element_type=jnp.float32) + bb_ref[...]
    out_ref[...] = o
    outb_ref[...] = o.astype(jnp.bfloat16)


def _tc_mlp1(fin, fout, x, parts, wa, ba, gamma, beta, wb, bb, scale):
    def part_spec(i):
        return pl.BlockSpec((1, N, fin), lambda g, i=i: (i, 0, 0))
    return pl.pallas_call(
        _tc_mlp1_body,
        grid=(1,),
        out_shape=[jax.ShapeDtypeStruct((N, fout), jnp.float32),
                   jax.ShapeDtypeStruct((N, fout), jnp.bfloat16)],
        in_specs=[
            pl.BlockSpec((N, fin), lambda g: (0, 0)),
            part_spec(0),
            part_spec(1),
            pl.BlockSpec(wa.shape, lambda g: (0, 0)),
            pl.BlockSpec(ba.shape, lambda g: (0, 0)),
            pl.BlockSpec(gamma.shape, lambda g: (0, 0)),
            pl.BlockSpec(beta.shape, lambda g: (0, 0)),
            pl.BlockSpec(wb.shape, lambda g: (0, 0)),
            pl.BlockSpec(bb.shape, lambda g: (0, 0)),
            pl.BlockSpec(memory_space=pltpu.SMEM),
        ],
        out_specs=[pl.BlockSpec((N, fout), lambda g: (0, 0)),
                   pl.BlockSpec((N, fout), lambda g: (0, 0))],
    )(x, parts, parts, wa, ba, gamma, beta, wb, bb, scale)


def _tc_mlp2_pool_body(x_ref, p0_ref, p1_ref, wa_ref, ba_ref, g_ref, be_ref,
                       wb_ref, bb_ref, batch_ref, wlin_ref, blin_ref,
                       scale_ref, out_ref):
    agg = (p0_ref[0] + p1_ref[0]).astype(jnp.float32)
    z = scale_ref[0, 0] * x_ref[...] + agg
    h = jnp.dot(z, wa_ref[...], preferred_element_type=jnp.float32) + ba_ref[...]
    mu = jnp.mean(h, axis=0, keepdims=True)
    d = h - mu
    var = jnp.mean(d * d, axis=0, keepdims=True)
    hn = d * lax.rsqrt(var + 1e-5) * g_ref[...] + be_ref[...]
    hr = jnp.maximum(hn, 0.0)
    h2 = jnp.dot(hr, wb_ref[...], preferred_element_type=jnp.float32) + bb_ref[...]
    gid = lax.broadcasted_iota(jnp.int32, (G, N), 0)
    oh = (gid == batch_ref[...]).astype(jnp.float32)
    pooled = jnp.dot(oh, h2, preferred_element_type=jnp.float32)
    counts = jnp.sum(oh, axis=1, keepdims=True)
    pm = pooled / jnp.maximum(counts, 1.0)
    logits = jnp.dot(pm, wlin_ref[...], preferred_element_type=jnp.float32)
    out_ref[...] = jax.nn.sigmoid(logits + blin_ref[...])


def _tc_mlp2_pool(fin, x, parts, wa, ba, gamma, beta, wb, bb, batch2d,
                  wlin, blin, scale):
    def part_spec(i):
        return pl.BlockSpec((1, N, fin), lambda g, i=i: (i, 0, 0))
    return pl.pallas_call(
        _tc_mlp2_pool_body,
        grid=(1,),
        out_shape=jax.ShapeDtypeStruct((G, 1), jnp.float32),
        in_specs=[
            pl.BlockSpec((N, fin), lambda g: (0, 0)),
            part_spec(0),
            part_spec(1),
            pl.BlockSpec(wa.shape, lambda g: (0, 0)),
            pl.BlockSpec(ba.shape, lambda g: (0, 0)),
            pl.BlockSpec(gamma.shape, lambda g: (0, 0)),
            pl.BlockSpec(beta.shape, lambda g: (0, 0)),
            pl.BlockSpec(wb.shape, lambda g: (0, 0)),
            pl.BlockSpec(bb.shape, lambda g: (0, 0)),
            pl.BlockSpec((1, N), lambda g: (0, 0)),
            pl.BlockSpec(wlin.shape, lambda g: (0, 0)),
            pl.BlockSpec(blin.shape, lambda g: (0, 0)),
            pl.BlockSpec(memory_space=pltpu.SMEM),
        ],
        out_specs=pl.BlockSpec((G, 1), lambda g: (0, 0)),
    )(x, parts, parts, wa, ba, gamma, beta, wb, bb, batch2d, wlin, blin, scale)


_sc_agg_128 = _make_sc_agg(128, 134)
_sc_agg_32 = _make_sc_agg(32, 126)


def kernel(x, edge_index, batch, W1a, b1a, gamma1, beta1, W1b, b1b, eps1,
           W2a, b2a, gamma2, beta2, W2b, b2b, eps2, Wlin, blin):
    srcp = edge_index[0].reshape(_NS * _CHUNKS, _K)
    dstp = edge_index[1].reshape(_NS * _CHUNKS, _K)

    batch2d = batch.reshape(1, N)
    se1 = (1.0 + eps1).reshape(1, 1).astype(jnp.float32)
    se2 = (1.0 + eps2).reshape(1, 1).astype(jnp.float32)

    xb = x.astype(jnp.bfloat16)
    z128 = jnp.zeros((_ROWS_PER_TILE, 128), jnp.bfloat16)
    z32 = jnp.zeros((_ROWS_PER_TILE, 32), jnp.bfloat16)
    parts1 = _sc_agg_128(xb, srcp, dstp, z128)
    h1, h1b = _tc_mlp1(128, 32, x, parts1,
                       W1a, b1a.reshape(1, -1), gamma1.reshape(1, -1),
                       beta1.reshape(1, -1), W1b, b1b.reshape(1, -1), se1)
    parts2 = _sc_agg_32(h1b, srcp, dstp, z32)
    return _tc_mlp2_pool(32, h1, parts2,
                         W2a, b2a.reshape(1, -1), gamma2.reshape(1, -1),
                         beta2.reshape(1, -1), W2b, b2b.reshape(1, -1),
                         batch2d, Wlin, blin.reshape(1, 1), se2)


# speedup vs baseline: 1.7201x; 1.0440x over previous
import functools

import jax
import jax.numpy as jnp
from jax import lax
from jax.experimental import pallas as pl
from jax.experimental.pallas import tpu as pltpu
from jax.experimental.pallas import tpu_sc as plsc

N = 10000
E = 320000
G = 64

_NC = 2
_NS = 16
_K = 80
_CHUNKS = 250
_ACC_ROWS = 10240
_ROWS_PER_TILE = _ACC_ROWS // _NS


def _make_sc_agg(feat, c0):
    c1 = _CHUNKS - c0
    assert c0 % 2 == 0 and c1 % 2 == 0
    mesh = plsc.VectorSubcoreMesh(core_axis_name="c", subcore_axis_name="s")

    @functools.partial(
        pl.kernel,
        mesh=mesh,
        compiler_params=pltpu.CompilerParams(use_tc_tiling_on_sc=False),
        out_type=jax.ShapeDtypeStruct((_NC, _ACC_ROWS, feat), jnp.bfloat16),
        scratch_types=[
            pltpu.VMEM((_K, feat), jnp.bfloat16),
            pltpu.VMEM((_K, feat), jnp.bfloat16),
            pltpu.VMEM((max(c0, _CHUNKS - c0), _K), jnp.int32),
            pltpu.VMEM((max(c0, _CHUNKS - c0), _K), jnp.int32),
            pltpu.VMEM_SHARED((_ACC_ROWS, feat), jnp.bfloat16),
            pltpu.SemaphoreType.DMA,
            pltpu.SemaphoreType.DMA,
        ],
    )
    def sc_agg(x_hbm, eidx_hbm, zeros_hbm, out_hbm, rows0, rows1,
               sidx_v, didx_v, acc, gs0, gs1):
        c = lax.axis_index("c")
        s = lax.axis_index("s")
        nq = _NS * _CHUNKS

        pltpu.sync_copy(zeros_hbm, acc.at[pl.ds(s * _ROWS_PER_TILE, _ROWS_PER_TILE)])
        plsc.subcore_barrier()

        def run(base, nchunks):
            pltpu.sync_copy(eidx_hbm.at[pl.ds(base, nchunks)],
                            sidx_v.at[pl.ds(0, nchunks)])
            pltpu.sync_copy(eidx_hbm.at[pl.ds(nq + base, nchunks)],
                            didx_v.at[pl.ds(0, nchunks)])

            pltpu.async_copy(x_hbm.at[sidx_v.at[0]], rows0, gs0)

            def pair_body(g, carry):
                j0 = 2 * g
                pltpu.async_copy(x_hbm.at[sidx_v.at[j0 + 1]], rows1, gs1)
                pltpu.make_async_copy(x_hbm.at[sidx_v.at[j0]], rows0, gs0).wait()
                pltpu.sync_copy(rows0, acc.at[didx_v.at[j0]], add=True)

                @pl.when(g + 1 < nchunks // 2)
                def _():
                    pltpu.async_copy(x_hbm.at[sidx_v.at[j0 + 2]], rows0, gs0)

                pltpu.make_async_copy(x_hbm.at[sidx_v.at[j0 + 1]], rows1,
                                      gs1).wait()
                pltpu.sync_copy(rows1, acc.at[didx_v.at[j0 + 1]], add=True)
                return carry

            lax.fori_loop(0, nchunks // 2, pair_body, 0)

        @pl.when(c == 0)
        def _():
            run(s * c0, c0)

        @pl.when(c == 1)
        def _():
            run(_NS * c0 + s * c1, c1)

        plsc.subcore_barrier()
        pltpu.sync_copy(
            acc.at[pl.ds(s * _ROWS_PER_TILE, _ROWS_PER_TILE)],
            out_hbm.at[c, pl.ds(s * _ROWS_PER_TILE, _ROWS_PER_TILE)],
        )

    return sc_agg


def _tc_mlp1_body(x_ref, p0_ref, p1_ref, wa_ref, ba_ref, g_ref, be_ref,
                  wb_ref, bb_ref, scale_ref, outb_ref):
    agg = (p0_ref[0] + p1_ref[0]).astype(jnp.float32)
    z = scale_ref[0, 0] * x_ref[...].astype(jnp.float32) + agg
    h = jnp.dot(z, wa_ref[...], preferred_element_type=jnp.float32) + ba_ref[...]
    mu = jnp.mean(h, axis=0, keepdims=True)
    d = h - mu
    var = jnp.mean(d * d, axis=0, keepdims=True)
    hn = d * lax.rsqrt(var + 1e-5) * g_ref[...] + be_ref[...]
    hr = jnp.maximum(hn, 0.0)
    o = jnp.dot(hr, wb_ref[...], preferred_element_type=jnp.float32) + bb_ref[...]
    outb_ref[...] = o.astype(jnp.bfloat16)


def _tc_mlp1(fin, fout, x, parts, wa, ba, gamma, beta, wb, bb, scale):
    def part_spec(i):
        return pl.BlockSpec((1, N, fin), lambda g, i=i: (i, 0, 0))
    return pl.pallas_call(
        _tc_mlp1_body,
        grid=(1,),
        out_shape=jax.ShapeDtypeStruct((N, fout), jnp.bfloat16),
        in_specs=[
            pl.BlockSpec((N, fin), lambda g: (0, 0)),
            part_spec(0),
            part_spec(1),
            pl.BlockSpec(wa.shape, lambda g: (0, 0)),
            pl.BlockSpec(ba.shape, lambda g: (0, 0)),
            pl.BlockSpec(gamma.shape, lambda g: (0, 0)),
            pl.BlockSpec(beta.shape, lambda g: (0, 0)),
            pl.BlockSpec(wb.shape, lambda g: (0, 0)),
            pl.BlockSpec(bb.shape, lambda g: (0, 0)),
            pl.BlockSpec(memory_space=pltpu.SMEM),
        ],
        out_specs=pl.BlockSpec((N, fout), lambda g: (0, 0)),
    )(x, parts, parts, wa, ba, gamma, beta, wb, bb, scale)


def _tc_mlp2_pool_body(x_ref, p0_ref, p1_ref, wa_ref, ba_ref, g_ref, be_ref,
                       wb_ref, bb_ref, batch_ref, wlin_ref, blin_ref,
                       scale_ref, out_ref):
    agg = (p0_ref[0] + p1_ref[0]).astype(jnp.float32)
    z = scale_ref[0, 0] * x_ref[...].astype(jnp.float32) + agg
    h = jnp.dot(z, wa_ref[...], preferred_element_type=jnp.float32) + ba_ref[...]
    mu = jnp.mean(h, axis=0, keepdims=True)
    d = h - mu
    var = jnp.mean(d * d, axis=0, keepdims=True)
    hn = d * lax.rsqrt(var + 1e-5) * g_ref[...] + be_ref[...]
    hr = jnp.maximum(hn, 0.0)
    h2 = jnp.dot(hr, wb_ref[...], preferred_element_type=jnp.float32) + bb_ref[...]
    gid = lax.broadcasted_iota(jnp.int32, (G, N), 0)
    oh = (gid == batch_ref[...]).astype(jnp.float32)
    pooled = jnp.dot(oh, h2, preferred_element_type=jnp.float32)
    counts = jnp.sum(oh, axis=1, keepdims=True)
    pm = pooled / jnp.maximum(counts, 1.0)
    logits = jnp.dot(pm, wlin_ref[...], preferred_element_type=jnp.float32)
    out_ref[...] = jax.nn.sigmoid(logits + blin_ref[...])


def _tc_mlp2_pool(fin, x, parts, wa, ba, gamma, beta, wb, bb, batch2d,
                  wlin, blin, scale):
    def part_spec(i):
        return pl.BlockSpec((1, N, fin), lambda g, i=i: (i, 0, 0))
    return pl.pallas_call(
        _tc_mlp2_pool_body,
        grid=(1,),
        out_shape=jax.ShapeDtypeStruct((G, 1), jnp.float32),
        in_specs=[
            pl.BlockSpec((N, fin), lambda g: (0, 0)),
            part_spec(0),
            part_spec(1),
            pl.BlockSpec(wa.shape, lambda g: (0, 0)),
            pl.BlockSpec(ba.shape, lambda g: (0, 0)),
            pl.BlockSpec(gamma.shape, lambda g: (0, 0)),
            pl.BlockSpec(beta.shape, lambda g: (0, 0)),
            pl.BlockSpec(wb.shape, lambda g: (0, 0)),
            pl.BlockSpec(bb.shape, lambda g: (0, 0)),
            pl.BlockSpec((1, N), lambda g: (0, 0)),
            pl.BlockSpec(wlin.shape, lambda g: (0, 0)),
            pl.BlockSpec(blin.shape, lambda g: (0, 0)),
            pl.BlockSpec(memory_space=pltpu.SMEM),
        ],
        out_specs=pl.BlockSpec((G, 1), lambda g: (0, 0)),
    )(x, parts, parts, wa, ba, gamma, beta, wb, bb, batch2d, wlin, blin, scale)


_sc_agg_128 = _make_sc_agg(128, 134)
_sc_agg_32 = _make_sc_agg(32, 126)


def kernel(x, edge_index, batch, W1a, b1a, gamma1, beta1, W1b, b1b, eps1,
           W2a, b2a, gamma2, beta2, W2b, b2b, eps2, Wlin, blin):
    eidx = edge_index.reshape(2 * _NS * _CHUNKS, _K)

    batch2d = batch.reshape(1, N)
    se1 = (1.0 + eps1).reshape(1, 1).astype(jnp.float32)
    se2 = (1.0 + eps2).reshape(1, 1).astype(jnp.float32)

    xb = x.astype(jnp.bfloat16)
    z128 = jnp.zeros((_ROWS_PER_TILE, 128), jnp.bfloat16)
    z32 = jnp.zeros((_ROWS_PER_TILE, 32), jnp.bfloat16)
    parts1 = _sc_agg_128(xb, eidx, z128)
    h1b = _tc_mlp1(128, 32, xb, parts1,
                   W1a, b1a.reshape(1, -1), gamma1.reshape(1, -1),
                   beta1.reshape(1, -1), W1b, b1b.reshape(1, -1), se1)
    parts2 = _sc_agg_32(h1b, eidx, z32)
    return _tc_mlp2_pool(32, h1b, parts2,
                         W2a, b2a.reshape(1, -1), gamma2.reshape(1, -1),
                         beta2.reshape(1, -1), W2b, b2b.reshape(1, -1),
                         batch2d, Wlin, blin.reshape(1, 1), se2)


# speedup vs baseline: 1.9095x; 1.1101x over previous
import functools

import jax
import jax.numpy as jnp
from jax import lax
from jax.experimental import pallas as pl
from jax.experimental.pallas import tpu as pltpu
from jax.experimental.pallas import tpu_sc as plsc

N = 10000
E = 320000
G = 64

_NC = 2
_NS = 16
_K = 80
_CHUNKS = 250
_ACC_ROWS = 10240
_ROWS_PER_TILE = _ACC_ROWS // _NS


def _make_sc_agg(feat, c0):
    c1 = _CHUNKS - c0
    assert c0 % 2 == 0 and c1 % 2 == 0
    mesh = plsc.VectorSubcoreMesh(core_axis_name="c", subcore_axis_name="s")

    @functools.partial(
        pl.kernel,
        mesh=mesh,
        compiler_params=pltpu.CompilerParams(use_tc_tiling_on_sc=False),
        out_type=jax.ShapeDtypeStruct((_NC, _ACC_ROWS, feat), jnp.bfloat16),
        scratch_types=[
            pltpu.VMEM((_K, feat), jnp.bfloat16),
            pltpu.VMEM((_K, feat), jnp.bfloat16),
            pltpu.VMEM((_K, feat), jnp.bfloat16),
            pltpu.VMEM((max(c0, _CHUNKS - c0), _K), jnp.int32),
            pltpu.VMEM((max(c0, _CHUNKS - c0), _K), jnp.int32),
            pltpu.VMEM_SHARED((_ACC_ROWS, feat), jnp.bfloat16),
            pltpu.SemaphoreType.DMA,
            pltpu.SemaphoreType.DMA,
            pltpu.SemaphoreType.DMA,
        ],
    )
    def sc_agg(x_hbm, eidx_hbm, zeros_hbm, out_hbm, rows0, rows1, rows2,
               sidx_v, didx_v, acc, gs0, gs1, gs2):
        c = lax.axis_index("c")
        s = lax.axis_index("s")
        nq = _NS * _CHUNKS

        pltpu.sync_copy(zeros_hbm, acc.at[pl.ds(s * _ROWS_PER_TILE, _ROWS_PER_TILE)])
        plsc.subcore_barrier()

        def run(base, nchunks):
            pltpu.sync_copy(eidx_hbm.at[pl.ds(base, nchunks)],
                            sidx_v.at[pl.ds(0, nchunks)])
            pltpu.sync_copy(eidx_hbm.at[pl.ds(nq + base, nchunks)],
                            didx_v.at[pl.ds(0, nchunks)])

            bufs = (rows0, rows1, rows2)
            gsem = (gs0, gs1, gs2)
            pltpu.async_copy(x_hbm.at[sidx_v.at[0]], rows0, gs0)
            pltpu.async_copy(x_hbm.at[sidx_v.at[1]], rows1, gs1)

            def triple_body(g, carry):
                for u in range(3):
                    j = 3 * g + u
                    v = (u + 2) % 3
                    pltpu.make_async_copy(x_hbm.at[sidx_v.at[j]], bufs[u],
                                          gsem[u]).wait()

                    @pl.when(j + 2 < nchunks)
                    def _():
                        pltpu.async_copy(x_hbm.at[sidx_v.at[j + 2]], bufs[v],
                                         gsem[v])

                    pltpu.sync_copy(bufs[u], acc.at[didx_v.at[j]], add=True)
                return carry

            lax.fori_loop(0, nchunks // 3, triple_body, 0)
            for i in range(nchunks % 3):
                j = (nchunks // 3) * 3 + i
                u = j % 3
                pltpu.make_async_copy(x_hbm.at[sidx_v.at[j]], bufs[u],
                                      gsem[u]).wait()
                pltpu.sync_copy(bufs[u], acc.at[didx_v.at[j]], add=True)

        @pl.when(c == 0)
        def _():
            run(s * c0, c0)

        @pl.when(c == 1)
        def _():
            run(_NS * c0 + s * c1, c1)

        plsc.subcore_barrier()
        pltpu.sync_copy(
            acc.at[pl.ds(s * _ROWS_PER_TILE, _ROWS_PER_TILE)],
            out_hbm.at[c, pl.ds(s * _ROWS_PER_TILE, _ROWS_PER_TILE)],
        )

    return sc_agg


def _tc_mlp1_body(x_ref, p0_ref, p1_ref, wa_ref, ba_ref, g_ref, be_ref,
                  wb_ref, bb_ref, scale_ref, outb_ref):
    agg = (p0_ref[0] + p1_ref[0]).astype(jnp.float32)
    z = scale_ref[0, 0] * x_ref[...].astype(jnp.float32) + agg
    h = jnp.dot(z, wa_ref[...], preferred_element_type=jnp.float32) + ba_ref[...]
    mu = jnp.mean(h, axis=0, keepdims=True)
    d = h - mu
    var = jnp.mean(d * d, axis=0, keepdims=True)
    hn = d * lax.rsqrt(var + 1e-5) * g_ref[...] + be_ref[...]
    hr = jnp.maximum(hn, 0.0)
    o = jnp.dot(hr, wb_ref[...], preferred_element_type=jnp.float32) + bb_ref[...]
    outb_ref[...] = o.astype(jnp.bfloat16)


def _tc_mlp1(fin, fout, x, parts, wa, ba, gamma, beta, wb, bb, scale):
    def part_spec(i):
        return pl.BlockSpec((1, N, fin), lambda g, i=i: (i, 0, 0))
    return pl.pallas_call(
        _tc_mlp1_body,
        grid=(1,),
        out_shape=jax.ShapeDtypeStruct((N, fout), jnp.bfloat16),
        in_specs=[
            pl.BlockSpec((N, fin), lambda g: (0, 0)),
            part_spec(0),
            part_spec(1),
            pl.BlockSpec(wa.shape, lambda g: (0, 0)),
            pl.BlockSpec(ba.shape, lambda g: (0, 0)),
            pl.BlockSpec(gamma.shape, lambda g: (0, 0)),
            pl.BlockSpec(beta.shape, lambda g: (0, 0)),
            pl.BlockSpec(wb.shape, lambda g: (0, 0)),
            pl.BlockSpec(bb.shape, lambda g: (0, 0)),
            pl.BlockSpec(memory_space=pltpu.SMEM),
        ],
        out_specs=pl.BlockSpec((N, fout), lambda g: (0, 0)),
    )(x, parts, parts, wa, ba, gamma, beta, wb, bb, scale)


def _tc_mlp2_pool_body(x_ref, p0_ref, p1_ref, wa_ref, ba_ref, g_ref, be_ref,
                       wb_ref, bb_ref, batch_ref, wlin_ref, blin_ref,
                       scale_ref, out_ref):
    agg = (p0_ref[0] + p1_ref[0]).astype(jnp.float32)
    z = scale_ref[0, 0] * x_ref[...].astype(jnp.float32) + agg
    h = jnp.dot(z, wa_ref[...], preferred_element_type=jnp.float32) + ba_ref[...]
    mu = jnp.mean(h, axis=0, keepdims=True)
    d = h - mu
    var = jnp.mean(d * d, axis=0, keepdims=True)
    hn = d * lax.rsqrt(var + 1e-5) * g_ref[...] + be_ref[...]
    hr = jnp.maximum(hn, 0.0)
    h2 = jnp.dot(hr, wb_ref[...], preferred_element_type=jnp.float32) + bb_ref[...]
    gid = lax.broadcasted_iota(jnp.int32, (G, N), 0)
    oh = (gid == batch_ref[...]).astype(jnp.float32)
    pooled = jnp.dot(oh, h2, preferred_element_type=jnp.float32)
    counts = jnp.sum(oh, axis=1, keepdims=True)
    pm = pooled / jnp.maximum(counts, 1.0)
    logits = jnp.dot(pm, wlin_ref[...], preferred_element_type=jnp.float32)
    out_ref[...] = jax.nn.sigmoid(logits + blin_ref[...])


def _tc_mlp2_pool(fin, x, parts, wa, ba, gamma, beta, wb, bb, batch2d,
                  wlin, blin, scale):
    def part_spec(i):
        return pl.BlockSpec((1, N, fin), lambda g, i=i: (i, 0, 0))
    return pl.pallas_call(
        _tc_mlp2_pool_body,
        grid=(1,),
        out_shape=jax.ShapeDtypeStruct((G, 1), jnp.float32),
        in_specs=[
            pl.BlockSpec((N, fin), lambda g: (0, 0)),
            part_spec(0),
            part_spec(1),
            pl.BlockSpec(wa.shape, lambda g: (0, 0)),
            pl.BlockSpec(ba.shape, lambda g: (0, 0)),
            pl.BlockSpec(gamma.shape, lambda g: (0, 0)),
            pl.BlockSpec(beta.shape, lambda g: (0, 0)),
            pl.BlockSpec(wb.shape, lambda g: (0, 0)),
            pl.BlockSpec(bb.shape, lambda g: (0, 0)),
            pl.BlockSpec((1, N), lambda g: (0, 0)),
            pl.BlockSpec(wlin.shape, lambda g: (0, 0)),
            pl.BlockSpec(blin.shape, lambda g: (0, 0)),
            pl.BlockSpec(memory_space=pltpu.SMEM),
        ],
        out_specs=pl.BlockSpec((G, 1), lambda g: (0, 0)),
    )(x, parts, parts, wa, ba, gamma, beta, wb, bb, batch2d, wlin, blin, scale)


_sc_agg_128 = _make_sc_agg(128, 126)
_sc_agg_32 = _make_sc_agg(32, 128)


def kernel(x, edge_index, batch, W1a, b1a, gamma1, beta1, W1b, b1b, eps1,
           W2a, b2a, gamma2, beta2, W2b, b2b, eps2, Wlin, blin):
    eidx = edge_index.reshape(2 * _NS * _CHUNKS, _K)

    batch2d = batch.reshape(1, N)
    se1 = (1.0 + eps1).reshape(1, 1).astype(jnp.float32)
    se2 = (1.0 + eps2).reshape(1, 1).astype(jnp.float32)

    xb = x.astype(jnp.bfloat16)
    z128 = jnp.zeros((_ROWS_PER_TILE, 128), jnp.bfloat16)
    z32 = jnp.zeros((_ROWS_PER_TILE, 32), jnp.bfloat16)
    parts1 = _sc_agg_128(xb, eidx, z128)
    h1b = _tc_mlp1(128, 32, xb, parts1,
                   W1a, b1a.reshape(1, -1), gamma1.reshape(1, -1),
                   beta1.reshape(1, -1), W1b, b1b.reshape(1, -1), se1)
    parts2 = _sc_agg_32(h1b, eidx, z32)
    return _tc_mlp2_pool(32, h1b, parts2,
                         W2a, b2a.reshape(1, -1), gamma2.reshape(1, -1),
                         beta2.reshape(1, -1), W2b, b2b.reshape(1, -1),
                         batch2d, Wlin, blin.reshape(1, 1), se2)


# speedup vs baseline: 2.1797x; 1.1415x over previous
import functools

import jax
import jax.numpy as jnp
from jax import lax
from jax.experimental import pallas as pl
from jax.experimental.pallas import tpu as pltpu
from jax.experimental.pallas import tpu_sc as plsc

N = 10000
E = 320000
G = 64

_NC = 2
_NS = 16
_K = 80
_CHUNKS = 250
_ACC_ROWS = 10240
_ROWS_PER_TILE = _ACC_ROWS // _NS


def _make_sc_agg(feat, c0):
    c1 = _CHUNKS - c0
    assert c0 % 2 == 0 and c1 % 2 == 0
    mesh = plsc.VectorSubcoreMesh(core_axis_name="c", subcore_axis_name="s")

    @functools.partial(
        pl.kernel,
        mesh=mesh,
        compiler_params=pltpu.CompilerParams(use_tc_tiling_on_sc=False),
        out_type=jax.ShapeDtypeStruct((_NC, _ACC_ROWS, feat), jnp.bfloat16),
        scratch_types=[
            pltpu.VMEM((_K, feat), jnp.bfloat16),
            pltpu.VMEM((_K, feat), jnp.bfloat16),
            pltpu.VMEM((_K, feat), jnp.bfloat16),
            pltpu.VMEM((_K, feat), jnp.bfloat16),
            pltpu.VMEM((max(c0, _CHUNKS - c0), _K), jnp.int32),
            pltpu.VMEM((max(c0, _CHUNKS - c0), _K), jnp.int32),
            pltpu.VMEM_SHARED((_ACC_ROWS, feat), jnp.bfloat16),
            pltpu.SemaphoreType.DMA,
            pltpu.SemaphoreType.DMA,
            pltpu.SemaphoreType.DMA,
            pltpu.SemaphoreType.DMA,
        ],
    )
    def sc_agg(x_hbm, eidx_hbm, zeros_hbm, out_hbm, rows0, rows1, rows2,
               rows3, sidx_v, didx_v, acc, gs0, gs1, gs2, gs3):
        c = lax.axis_index("c")
        s = lax.axis_index("s")
        nq = _NS * _CHUNKS

        pltpu.sync_copy(zeros_hbm, acc.at[pl.ds(s * _ROWS_PER_TILE, _ROWS_PER_TILE)])
        plsc.subcore_barrier()

        def run(base, nchunks):
            pltpu.sync_copy(eidx_hbm.at[pl.ds(base, nchunks)],
                            sidx_v.at[pl.ds(0, nchunks)])
            pltpu.sync_copy(eidx_hbm.at[pl.ds(nq + base, nchunks)],
                            didx_v.at[pl.ds(0, nchunks)])

            bufs = (rows0, rows1, rows2, rows3)
            gsem = (gs0, gs1, gs2, gs3)
            pltpu.async_copy(x_hbm.at[sidx_v.at[0]], rows0, gs0)
            pltpu.async_copy(x_hbm.at[sidx_v.at[1]], rows1, gs1)
            pltpu.async_copy(x_hbm.at[sidx_v.at[2]], rows2, gs2)

            def quad_body(g, carry):
                for u in range(4):
                    j = 4 * g + u
                    v = (u + 3) % 4
                    pltpu.make_async_copy(x_hbm.at[sidx_v.at[j]], bufs[u],
                                          gsem[u]).wait()

                    @pl.when(j + 3 < nchunks)
                    def _():
                        pltpu.async_copy(x_hbm.at[sidx_v.at[j + 3]], bufs[v],
                                         gsem[v])

                    pltpu.sync_copy(bufs[u], acc.at[didx_v.at[j]], add=True)
                return carry

            lax.fori_loop(0, nchunks // 4, quad_body, 0)
            for i in range(nchunks % 4):
                j = (nchunks // 4) * 4 + i
                u = j % 4
                pltpu.make_async_copy(x_hbm.at[sidx_v.at[j]], bufs[u],
                                      gsem[u]).wait()
                pltpu.sync_copy(bufs[u], acc.at[didx_v.at[j]], add=True)

        @pl.when(c == 0)
        def _():
            run(s * c0, c0)

        @pl.when(c == 1)
        def _():
            run(_NS * c0 + s * c1, c1)

        plsc.subcore_barrier()
        pltpu.sync_copy(
            acc.at[pl.ds(s * _ROWS_PER_TILE, _ROWS_PER_TILE)],
            out_hbm.at[c, pl.ds(s * _ROWS_PER_TILE, _ROWS_PER_TILE)],
        )

    return sc_agg


def _tc_mlp1_body(x_ref, p0_ref, p1_ref, wa_ref, ba_ref, g_ref, be_ref,
                  wb_ref, bb_ref, scale_ref, outb_ref):
    agg = (p0_ref[0] + p1_ref[0]).astype(jnp.float32)
    z = scale_ref[0, 0] * x_ref[...].astype(jnp.float32) + agg
    h = jnp.dot(z, wa_ref[...], preferred_element_type=jnp.float32) + ba_ref[...]
    mu = jnp.mean(h, axis=0, keepdims=True)
    d = h - mu
    var = jnp.mean(d * d, axis=0, keepdims=True)
    hn = d * lax.rsqrt(var + 1e-5) * g_ref[...] + be_ref[...]
    hr = jnp.maximum(hn, 0.0)
    o = jnp.dot(hr, wb_ref[...], preferred_element_type=jnp.float32) + bb_ref[...]
    outb_ref[...] = o.astype(jnp.bfloat16)


def _tc_mlp1(fin, fout, x, parts, wa, ba, gamma, beta, wb, bb, scale):
    def part_spec(i):
        return pl.BlockSpec((1, N, fin), lambda g, i=i: (i, 0, 0))
    return pl.pallas_call(
        _tc_mlp1_body,
        grid=(1,),
        out_shape=jax.ShapeDtypeStruct((N, fout), jnp.bfloat16),
        in_specs=[
            pl.BlockSpec((N, fin), lambda g: (0, 0)),
            part_spec(0),
            part_spec(1),
            pl.BlockSpec(wa.shape, lambda g: (0, 0)),
            pl.BlockSpec(ba.shape, lambda g: (0, 0)),
            pl.BlockSpec(gamma.shape, lambda g: (0, 0)),
            pl.BlockSpec(beta.shape, lambda g: (0, 0)),
            pl.BlockSpec(wb.shape, lambda g: (0, 0)),
            pl.BlockSpec(bb.shape, lambda g: (0, 0)),
            pl.BlockSpec(memory_space=pltpu.SMEM),
        ],
        out_specs=pl.BlockSpec((N, fout), lambda g: (0, 0)),
    )(x, parts, parts, wa, ba, gamma, beta, wb, bb, scale)


def _tc_mlp2_pool_body(x_ref, p0_ref, p1_ref, wa_ref, ba_ref, g_ref, be_ref,
                       wb_ref, bb_ref, batch_ref, wlin_ref, blin_ref,
                       scale_ref, out_ref):
    agg = (p0_ref[0] + p1_ref[0]).astype(jnp.float32)
    z = scale_ref[0, 0] * x_ref[...].astype(jnp.float32) + agg
    h = jnp.dot(z, wa_ref[...], preferred_element_type=jnp.float32) + ba_ref[...]
    mu = jnp.mean(h, axis=0, keepdims=True)
    d = h - mu
    var = jnp.mean(d * d, axis=0, keepdims=True)
    hn = d * lax.rsqrt(var + 1e-5) * g_ref[...] + be_ref[...]
    hr = jnp.maximum(hn, 0.0)
    h2 = jnp.dot(hr, wb_ref[...], preferred_element_type=jnp.float32) + bb_ref[...]
    gid = lax.broadcasted_iota(jnp.int32, (G, N), 0)
    oh = (gid == batch_ref[...]).astype(jnp.float32)
    pooled = jnp.dot(oh, h2, preferred_element_type=jnp.float32)
    counts = jnp.sum(oh, axis=1, keepdims=True)
    pm = pooled / jnp.maximum(counts, 1.0)
    logits = jnp.dot(pm, wlin_ref[...], preferred_element_type=jnp.float32)
    out_ref[...] = jax.nn.sigmoid(logits + blin_ref[...])


def _tc_mlp2_pool(fin, x, parts, wa, ba, gamma, beta, wb, bb, batch2d,
                  wlin, blin, scale):
    def part_spec(i):
        return pl.BlockSpec((1, N, fin), lambda g, i=i: (i, 0, 0))
    return pl.pallas_call(
        _tc_mlp2_pool_body,
        grid=(1,),
        out_shape=jax.ShapeDtypeStruct((G, 1), jnp.float32),
        in_specs=[
            pl.BlockSpec((N, fin), lambda g: (0, 0)),
            part_spec(0),
            part_spec(1),
            pl.BlockSpec(wa.shape, lambda g: (0, 0)),
            pl.BlockSpec(ba.shape, lambda g: (0, 0)),
            pl.BlockSpec(gamma.shape, lambda g: (0, 0)),
            pl.BlockSpec(beta.shape, lambda g: (0, 0)),
            pl.BlockSpec(wb.shape, lambda g: (0, 0)),
            pl.BlockSpec(bb.shape, lambda g: (0, 0)),
            pl.BlockSpec((1, N), lambda g: (0, 0)),
            pl.BlockSpec(wlin.shape, lambda g: (0, 0)),
            pl.BlockSpec(blin.shape, lambda g: (0, 0)),
            pl.BlockSpec(memory_space=pltpu.SMEM),
        ],
        out_specs=pl.BlockSpec((G, 1), lambda g: (0, 0)),
    )(x, parts, parts, wa, ba, gamma, beta, wb, bb, batch2d, wlin, blin, scale)


_sc_agg_128 = _make_sc_agg(128, 126)
_sc_agg_32 = _make_sc_agg(32, 126)


def kernel(x, edge_index, batch, W1a, b1a, gamma1, beta1, W1b, b1b, eps1,
           W2a, b2a, gamma2, beta2, W2b, b2b, eps2, Wlin, blin):
    eidx = edge_index.reshape(2 * _NS * _CHUNKS, _K)

    batch2d = batch.reshape(1, N)
    se1 = (1.0 + eps1).reshape(1, 1).astype(jnp.float32)
    se2 = (1.0 + eps2).reshape(1, 1).astype(jnp.float32)

    xb = x.astype(jnp.bfloat16)
    z128 = jnp.zeros((_ROWS_PER_TILE, 128), jnp.bfloat16)
    z32 = jnp.zeros((_ROWS_PER_TILE, 32), jnp.bfloat16)
    parts1 = _sc_agg_128(xb, eidx, z128)
    h1b = _tc_mlp1(128, 32, xb, parts1,
                   W1a, b1a.reshape(1, -1), gamma1.reshape(1, -1),
                   beta1.reshape(1, -1), W1b, b1b.reshape(1, -1), se1)
    parts2 = _sc_agg_32(h1b, eidx, z32)
    return _tc_mlp2_pool(32, h1b, parts2,
                         W2a, b2a.reshape(1, -1), gamma2.reshape(1, -1),
                         beta2.reshape(1, -1), W2b, b2b.reshape(1, -1),
                         batch2d, Wlin, blin.reshape(1, 1), se2)


# speedup vs baseline: 2.3384x; 1.0728x over previous
import functools

import jax
import jax.numpy as jnp
from jax import lax
from jax.experimental import pallas as pl
from jax.experimental.pallas import tpu as pltpu
from jax.experimental.pallas import tpu_sc as plsc

N = 10000
E = 320000
G = 64

_NC = 2
_NS = 16
_K = 80
_CHUNKS = 250
_ACC_ROWS = 10240
_ROWS_PER_TILE = _ACC_ROWS // _NS


def _make_sc_agg(feat, c0):
    c1 = _CHUNKS - c0
    assert c0 % 2 == 0 and c1 % 2 == 0
    mesh = plsc.VectorSubcoreMesh(core_axis_name="c", subcore_axis_name="s")

    @functools.partial(
        pl.kernel,
        mesh=mesh,
        compiler_params=pltpu.CompilerParams(use_tc_tiling_on_sc=False),
        out_type=jax.ShapeDtypeStruct((_NC, _ACC_ROWS, feat), jnp.bfloat16),
        scratch_types=[
            pltpu.VMEM((_K, feat), jnp.bfloat16),
            pltpu.VMEM((_K, feat), jnp.bfloat16),
            pltpu.VMEM((_K, feat), jnp.bfloat16),
            pltpu.VMEM((_K, feat), jnp.bfloat16),
            pltpu.VMEM((_K, feat), jnp.bfloat16),
            pltpu.VMEM((_K, feat), jnp.bfloat16),
            pltpu.VMEM((max(c0, _CHUNKS - c0), _K), jnp.int32),
            pltpu.VMEM((max(c0, _CHUNKS - c0), _K), jnp.int32),
            pltpu.VMEM_SHARED((_ACC_ROWS, feat), jnp.bfloat16),
            pltpu.SemaphoreType.DMA,
            pltpu.SemaphoreType.DMA,
            pltpu.SemaphoreType.DMA,
            pltpu.SemaphoreType.DMA,
            pltpu.SemaphoreType.DMA,
            pltpu.SemaphoreType.DMA,
        ],
    )
    def sc_agg(x_hbm, eidx_hbm, zeros_hbm, out_hbm, rows0, rows1, rows2,
               rows3, rows4, rows5, sidx_v, didx_v, acc, gs0, gs1, gs2, gs3,
               gs4, gs5):
        c = lax.axis_index("c")
        s = lax.axis_index("s")
        nq = _NS * _CHUNKS

        pltpu.sync_copy(zeros_hbm, acc.at[pl.ds(s * _ROWS_PER_TILE, _ROWS_PER_TILE)])
        plsc.subcore_barrier()

        def run(base, nchunks):
            pltpu.sync_copy(eidx_hbm.at[pl.ds(base, nchunks)],
                            sidx_v.at[pl.ds(0, nchunks)])
            pltpu.sync_copy(eidx_hbm.at[pl.ds(nq + base, nchunks)],
                            didx_v.at[pl.ds(0, nchunks)])

            bufs = (rows0, rows1, rows2, rows3, rows4, rows5)
            gsem = (gs0, gs1, gs2, gs3, gs4, gs5)
            ns = 6
            for u in range(ns - 1):
                pltpu.async_copy(x_hbm.at[sidx_v.at[u]], bufs[u], gsem[u])

            def ring_body(g, carry):
                for u in range(ns):
                    j = ns * g + u
                    v = (u + ns - 1) % ns
                    pltpu.make_async_copy(x_hbm.at[sidx_v.at[j]], bufs[u],
                                          gsem[u]).wait()

                    @pl.when(j + ns - 1 < nchunks)
                    def _():
                        pltpu.async_copy(x_hbm.at[sidx_v.at[j + ns - 1]],
                                         bufs[v], gsem[v])

                    pltpu.sync_copy(bufs[u], acc.at[didx_v.at[j]], add=True)
                return carry

            lax.fori_loop(0, nchunks // ns, ring_body, 0)
            for i in range(nchunks % ns):
                j = (nchunks // ns) * ns + i
                u = j % ns
                pltpu.make_async_copy(x_hbm.at[sidx_v.at[j]], bufs[u],
                                      gsem[u]).wait()
                pltpu.sync_copy(bufs[u], acc.at[didx_v.at[j]], add=True)

        @pl.when(c == 0)
        def _():
            run(s * c0, c0)

        @pl.when(c == 1)
        def _():
            run(_NS * c0 + s * c1, c1)

        plsc.subcore_barrier()
        pltpu.sync_copy(
            acc.at[pl.ds(s * _ROWS_PER_TILE, _ROWS_PER_TILE)],
            out_hbm.at[c, pl.ds(s * _ROWS_PER_TILE, _ROWS_PER_TILE)],
        )

    return sc_agg


def _tc_mlp1_body(x_ref, p0_ref, p1_ref, wa_ref, ba_ref, g_ref, be_ref,
                  wb_ref, bb_ref, scale_ref, outb_ref):
    agg = (p0_ref[0] + p1_ref[0]).astype(jnp.float32)
    z = scale_ref[0, 0] * x_ref[...].astype(jnp.float32) + agg
    h = jnp.dot(z, wa_ref[...], preferred_element_type=jnp.float32) + ba_ref[...]
    mu = jnp.mean(h, axis=0, keepdims=True)
    d = h - mu
    var = jnp.mean(d * d, axis=0, keepdims=True)
    hn = d * lax.rsqrt(var + 1e-5) * g_ref[...] + be_ref[...]
    hr = jnp.maximum(hn, 0.0)
    o = jnp.dot(hr, wb_ref[...], preferred_element_type=jnp.float32) + bb_ref[...]
    outb_ref[...] = o.astype(jnp.bfloat16)


def _tc_mlp1(fin, fout, x, parts, wa, ba, gamma, beta, wb, bb, scale):
    def part_spec(i):
        return pl.BlockSpec((1, N, fin), lambda g, i=i: (i, 0, 0))
    return pl.pallas_call(
        _tc_mlp1_body,
        grid=(1,),
        out_shape=jax.ShapeDtypeStruct((N, fout), jnp.bfloat16),
        in_specs=[
            pl.BlockSpec((N, fin), lambda g: (0, 0)),
            part_spec(0),
            part_spec(1),
            pl.BlockSpec(wa.shape, lambda g: (0, 0)),
            pl.BlockSpec(ba.shape, lambda g: (0, 0)),
            pl.BlockSpec(gamma.shape, lambda g: (0, 0)),
            pl.BlockSpec(beta.shape, lambda g: (0, 0)),
            pl.BlockSpec(wb.shape, lambda g: (0, 0)),
            pl.BlockSpec(bb.shape, lambda g: (0, 0)),
            pl.BlockSpec(memory_space=pltpu.SMEM),
        ],
        out_specs=pl.BlockSpec((N, fout), lambda g: (0, 0)),
    )(x, parts, parts, wa, ba, gamma, beta, wb, bb, scale)


def _tc_mlp2_pool_body(x_ref, p0_ref, p1_ref, wa_ref, ba_ref, g_ref, be_ref,
                       wb_ref, bb_ref, batch_ref, wlin_ref, blin_ref,
                       scale_ref, out_ref):
    agg = (p0_ref[0] + p1_ref[0]).astype(jnp.float32)
    z = scale_ref[0, 0] * x_ref[...].astype(jnp.float32) + agg
    h = jnp.dot(z, wa_ref[...], preferred_element_type=jnp.float32) + ba_ref[...]
    mu = jnp.mean(h, axis=0, keepdims=True)
    d = h - mu
    var = jnp.mean(d * d, axis=0, keepdims=True)
    hn = d * lax.rsqrt(var + 1e-5) * g_ref[...] + be_ref[...]
    hr = jnp.maximum(hn, 0.0)
    h2 = jnp.dot(hr, wb_ref[...], preferred_element_type=jnp.float32) + bb_ref[...]
    gid = lax.broadcasted_iota(jnp.int32, (G, N), 0)
    oh = (gid == batch_ref[...]).astype(jnp.float32)
    pooled = jnp.dot(oh, h2, preferred_element_type=jnp.float32)
    counts = jnp.sum(oh, axis=1, keepdims=True)
    pm = pooled / jnp.maximum(counts, 1.0)
    logits = jnp.dot(pm, wlin_ref[...], preferred_element_type=jnp.float32)
    out_ref[...] = jax.nn.sigmoid(logits + blin_ref[...])


def _tc_mlp2_pool(fin, x, parts, wa, ba, gamma, beta, wb, bb, batch2d,
                  wlin, blin, scale):
    def part_spec(i):
        return pl.BlockSpec((1, N, fin), lambda g, i=i: (i, 0, 0))
    return pl.pallas_call(
        _tc_mlp2_pool_body,
        grid=(1,),
        out_shape=jax.ShapeDtypeStruct((G, 1), jnp.float32),
        in_specs=[
            pl.BlockSpec((N, fin), lambda g: (0, 0)),
            part_spec(0),
            part_spec(1),
            pl.BlockSpec(wa.shape, lambda g: (0, 0)),
            pl.BlockSpec(ba.shape, lambda g: (0, 0)),
            pl.BlockSpec(gamma.shape, lambda g: (0, 0)),
            pl.BlockSpec(beta.shape, lambda g: (0, 0)),
            pl.BlockSpec(wb.shape, lambda g: (0, 0)),
            pl.BlockSpec(bb.shape, lambda g: (0, 0)),
            pl.BlockSpec((1, N), lambda g: (0, 0)),
            pl.BlockSpec(wlin.shape, lambda g: (0, 0)),
            pl.BlockSpec(blin.shape, lambda g: (0, 0)),
            pl.BlockSpec(memory_space=pltpu.SMEM),
        ],
        out_specs=pl.BlockSpec((G, 1), lambda g: (0, 0)),
    )(x, parts, parts, wa, ba, gamma, beta, wb, bb, batch2d, wlin, blin, scale)


_sc_agg_128 = _make_sc_agg(128, 126)
_sc_agg_32 = _make_sc_agg(32, 126)


def kernel(x, edge_index, batch, W1a, b1a, gamma1, beta1, W1b, b1b, eps1,
           W2a, b2a, gamma2, beta2, W2b, b2b, eps2, Wlin, blin):
    eidx = edge_index.reshape(2 * _NS * _CHUNKS, _K)

    batch2d = batch.reshape(1, N)
    se1 = (1.0 + eps1).reshape(1, 1).astype(jnp.float32)
    se2 = (1.0 + eps2).reshape(1, 1).astype(jnp.float32)

    xb = x.astype(jnp.bfloat16)
    z128 = jnp.zeros((_ROWS_PER_TILE, 128), jnp.bfloat16)
    z32 = jnp.zeros((_ROWS_PER_TILE, 32), jnp.bfloat16)
    parts1 = _sc_agg_128(xb, eidx, z128)
    h1b = _tc_mlp1(128, 32, xb, parts1,
                   W1a, b1a.reshape(1, -1), gamma1.reshape(1, -1),
                   beta1.reshape(1, -1), W1b, b1b.reshape(1, -1), se1)
    parts2 = _sc_agg_32(h1b, eidx, z32)
    return _tc_mlp2_pool(32, h1b, parts2,
                         W2a, b2a.reshape(1, -1), gamma2.reshape(1, -1),
                         beta2.reshape(1, -1), W2b, b2b.reshape(1, -1),
                         batch2d, Wlin, blin.reshape(1, 1), se2)
